# baseline (speedup 1.0000x reference)
import numpy as np

import jax
import jax.numpy as jnp
from jax.experimental import pallas as pl
from jax.experimental.pallas import tpu as pltpu

_EPS = 1e-6
_S = 0.025
_ALPHA = 0.98
_DELTA = 2.0
_R = 0.5

_TILE = 128
_TC = 4096


def _pcen_kernel(l_ref, x_ref, o_ref, carry_ref):
    f32 = jnp.float32
    L = l_ref[...]
    dcol = L[:, 0:1] * ((1.0 - _S) / _S)
    cdecay = (1.0 - _S) ** _TILE
    neg_droot = -(_DELTA ** _R)

    @pl.when(pl.program_id(1) == 0)
    def _init():
        carry_ref[...] = x_ref[0, 0:1, :]

    carry = carry_ref[...]
    for k in range(_TC // _TILE):
        v = x_ref[0, k * _TILE:(k + 1) * _TILE, :]
        p = jax.lax.dot(
            L, v, preferred_element_type=f32,
            precision=jax.lax.Precision.DEFAULT,
        )
        m_eps = (p + dcol * carry) + _EPS
        carry = p[_TILE - 1:_TILE, :] + cdecay * carry
        w = jnp.exp2(jnp.log2(m_eps) * (-_ALPHA))
        y = v * w + _DELTA
        o_ref[0, k * _TILE:(k + 1) * _TILE, :] = (
            jax.lax.rsqrt(y) * y + neg_droot
        )
    carry_ref[...] = carry


@jax.jit
def kernel(x):
    B, T, F = x.shape
    idx = np.arange(_TILE)
    diff = idx[:, None] - idx[None, :]
    lmat = jnp.asarray(
        np.where(diff >= 0, _S * (1.0 - _S) ** diff, 0.0), dtype=jnp.float32
    )
    grid = (B, T // _TC)
    return pl.pallas_call(
        _pcen_kernel,
        grid=grid,
        in_specs=[
            pl.BlockSpec((_TILE, _TILE), lambda b, t: (0, 0)),
            pl.BlockSpec((1, _TC, F), lambda b, t: (b, t, 0)),
        ],
        out_specs=pl.BlockSpec((1, _TC, F), lambda b, t: (b, t, 0)),
        out_shape=jax.ShapeDtypeStruct((B, T, F), x.dtype),
        scratch_shapes=[pltpu.VMEM((1, F), jnp.float32)],
        compiler_params=pltpu.CompilerParams(
            dimension_semantics=("parallel", "arbitrary"),
        ),
    )(lmat, x)


# speedup vs baseline: 1.1876x; 1.1876x over previous
import numpy as np

import jax
import jax.numpy as jnp
from jax.experimental import pallas as pl
from jax.experimental.pallas import tpu as pltpu

_EPS = 1e-6
_S = 0.025
_ALPHA = 0.98
_DELTA = 2.0
_R = 0.5

_TILE = 128
_TC = 8192


def _pcen_kernel(l_ref, x_ref, o_ref, carry_ref):
    f32 = jnp.float32
    L = l_ref[...]
    dcol = L[:, 0:1] * ((1.0 - _S) / _S)
    cdecay = (1.0 - _S) ** _TILE
    neg_droot = -(_DELTA ** _R)

    @pl.when(pl.program_id(1) == 0)
    def _init():
        carry_ref[...] = x_ref[0, 0:1, :]

    carry = carry_ref[...]
    for k in range(_TC // _TILE):
        v = x_ref[0, k * _TILE:(k + 1) * _TILE, :]
        p = jax.lax.dot(
            L, v, preferred_element_type=f32,
            precision=jax.lax.Precision.DEFAULT,
        )
        m_eps = (p + dcol * carry) + _EPS
        carry = p[_TILE - 1:_TILE, :] + cdecay * carry
        w = jnp.exp2(jnp.log2(m_eps) * (-_ALPHA))
        y = v * w + _DELTA
        o_ref[0, k * _TILE:(k + 1) * _TILE, :] = (
            jax.lax.rsqrt(y) * y + neg_droot
        )
    carry_ref[...] = carry


@jax.jit
def kernel(x):
    B, T, F = x.shape
    idx = np.arange(_TILE)
    diff = idx[:, None] - idx[None, :]
    lmat = jnp.asarray(
        np.where(diff >= 0, _S * (1.0 - _S) ** diff, 0.0), dtype=jnp.float32
    )
    grid = (B, T // _TC)
    return pl.pallas_call(
        _pcen_kernel,
        grid=grid,
        in_specs=[
            pl.BlockSpec((_TILE, _TILE), lambda b, t: (0, 0)),
            pl.BlockSpec((1, _TC, F), lambda b, t: (b, t, 0)),
        ],
        out_specs=pl.BlockSpec((1, _TC, F), lambda b, t: (b, t, 0)),
        out_shape=jax.ShapeDtypeStruct((B, T, F), x.dtype),
        scratch_shapes=[pltpu.VMEM((1, F), jnp.float32)],
        compiler_params=pltpu.CompilerParams(
            dimension_semantics=("parallel", "arbitrary"),
        ),
    )(lmat, x)
